# Initial kernel scaffold; baseline (speedup 1.0000x reference)
import jax
import jax.numpy as jnp
from jax.experimental import pallas as pl


def kernel(latent, codebook):
    raise NotImplementedError("write your pallas kernel here")


# speedup vs baseline: 29.1816x; 29.1816x over previous
import functools

import jax
import jax.numpy as jnp
from jax.experimental import pallas as pl
from jax.experimental.pallas import tpu as pltpu

_E = 8
_D = 4

_ROWS = 4096
_COLS = 4096
_BLOCK_ROWS = 32
_CHUNK = 128


def _vq_tc_kernel(cb_ref, x_ref, o_ref):
    lane = jax.lax.broadcasted_iota(jnp.int32, (1, _CHUNK), 1)
    d = lane & 3
    md0 = d == 0
    md1 = d == 1
    md2 = d == 2

    for c in range(0, _COLS, _CHUNK):
        x = x_ref[:, c:c + _CHUNK]

        r = {s: jnp.roll(x, s, axis=1) for s in (-3, -2, -1, 1, 2, 3)}
        y0 = jnp.where(md0, x, jnp.where(md1, r[1], jnp.where(md2, r[2], r[3])))
        y1 = jnp.where(md0, r[-1], jnp.where(md1, x, jnp.where(md2, r[1], r[2])))
        y2 = jnp.where(md0, r[-2], jnp.where(md1, r[-1], jnp.where(md2, x, r[1])))
        y3 = jnp.where(md0, r[-3], jnp.where(md1, r[-2], jnp.where(md2, r[-1], x)))

        xx = ((y0 * y0 + y1 * y1) + y2 * y2) + y3 * y3

        y0b = y0.astype(jnp.bfloat16).astype(jnp.float32)
        y1b = y1.astype(jnp.bfloat16).astype(jnp.float32)
        y2b = y2.astype(jnp.bfloat16).astype(jnp.float32)
        y3b = y3.astype(jnp.bfloat16).astype(jnp.float32)

        best = None
        q = None
        for j in range(_E):
            c0 = cb_ref[j, 0]
            c1 = cb_ref[j, 1]
            c2 = cb_ref[j, 2]
            c3 = cb_ref[j, 3]
            c2sum = cb_ref[j, 4]
            dot = ((y0b * c0 + y1b * c1) + y2b * c2) + y3b * c3
            s = (xx + c2sum) - (dot + dot)
            cj = jnp.where(md0, c0, jnp.where(md1, c1, jnp.where(md2, c2, c3)))
            cj = jnp.broadcast_to(cj, x.shape)
            if best is None:
                best, q = s, cj
            else:
                m = s < best
                best = jnp.minimum(best, s)
                q = jnp.where(m, cj, q)
        o_ref[:, c:c + _CHUNK] = q


@jax.jit
def kernel(latent, codebook):
    xf = latent.reshape(_ROWS, _COLS)
    c2 = jnp.sum(codebook ** 2, axis=-1)
    cb_r = codebook.astype(jnp.bfloat16).astype(jnp.float32)
    cb_aug = jnp.concatenate([cb_r, c2[:, None]], axis=1)

    grid = (_ROWS // _BLOCK_ROWS,)
    q = pl.pallas_call(
        _vq_tc_kernel,
        grid=grid,
        in_specs=[
            pl.BlockSpec(memory_space=pltpu.SMEM),
            pl.BlockSpec((_BLOCK_ROWS, _COLS), lambda i: (i, 0)),
        ],
        out_specs=pl.BlockSpec((_BLOCK_ROWS, _COLS), lambda i: (i, 0)),
        out_shape=jax.ShapeDtypeStruct((_ROWS, _COLS), jnp.float32),
        compiler_params=pltpu.CompilerParams(
            dimension_semantics=("arbitrary",),
        ),
    )(cb_aug, xf)
    qr = q.reshape(latent.shape)
    return (qr, qr)


# speedup vs baseline: 30.1898x; 1.0345x over previous
import functools

import jax
import jax.numpy as jnp
from jax.experimental import pallas as pl
from jax.experimental.pallas import tpu as pltpu

_E = 8
_D = 4

_ROWS = 4096
_COLS = 4096
_BLOCK_ROWS = 32
_CHUNK = 128


def _vq_tc_kernel(cb_ref, x_ref, o_ref):
    lane = jax.lax.broadcasted_iota(jnp.int32, (1, _CHUNK), 1)
    d = lane & 3
    md0 = d == 0
    md1 = d == 1
    md2 = d == 2

    for c in range(0, _COLS, _CHUNK):
        xr = x_ref[:, c:c + _CHUNK]
        x = xr.astype(jnp.bfloat16).astype(jnp.float32)

        r = {s: jnp.roll(x, s, axis=1) for s in (-3, -2, -1, 1, 2, 3)}
        y0 = jnp.where(md0, x, jnp.where(md1, r[1], jnp.where(md2, r[2], r[3])))
        y1 = jnp.where(md0, r[-1], jnp.where(md1, x, jnp.where(md2, r[1], r[2])))
        y2 = jnp.where(md0, r[-2], jnp.where(md1, r[-1], jnp.where(md2, x, r[1])))
        y3 = jnp.where(md0, r[-3], jnp.where(md1, r[-2], jnp.where(md2, r[-1], x)))

        best = None
        q = None
        for j in range(_E):
            c0 = cb_ref[j, 0]
            c1 = cb_ref[j, 1]
            c2 = cb_ref[j, 2]
            c3 = cb_ref[j, 3]
            c2sum = cb_ref[j, 4]
            dot = ((y0 * c0 + y1 * c1) + y2 * c2) + y3 * c3
            s = c2sum - (dot + dot)
            cj = jnp.where(md0, c0, jnp.where(md1, c1, jnp.where(md2, c2, c3)))
            cj = jnp.broadcast_to(cj, x.shape)
            if best is None:
                best, q = s, cj
            else:
                m = s < best
                best = jnp.minimum(best, s)
                q = jnp.where(m, cj, q)
        o_ref[:, c:c + _CHUNK] = q


@jax.jit
def kernel(latent, codebook):
    xf = latent.reshape(_ROWS, _COLS)
    c2 = jnp.sum(codebook ** 2, axis=-1)
    cb_r = codebook.astype(jnp.bfloat16).astype(jnp.float32)
    cb_aug = jnp.concatenate([cb_r, c2[:, None]], axis=1)

    grid = (_ROWS // _BLOCK_ROWS,)
    q = pl.pallas_call(
        _vq_tc_kernel,
        grid=grid,
        in_specs=[
            pl.BlockSpec(memory_space=pltpu.SMEM),
            pl.BlockSpec((_BLOCK_ROWS, _COLS), lambda i: (i, 0)),
        ],
        out_specs=pl.BlockSpec((_BLOCK_ROWS, _COLS), lambda i: (i, 0)),
        out_shape=jax.ShapeDtypeStruct((_ROWS, _COLS), jnp.float32),
        compiler_params=pltpu.CompilerParams(
            dimension_semantics=("arbitrary",),
        ),
    )(cb_aug, xf)
    qr = q.reshape(latent.shape)
    return (qr, qr)
